# Initial kernel scaffold; baseline (speedup 1.0000x reference)
import jax
import jax.numpy as jnp
from jax.experimental import pallas as pl


def kernel(x):
    raise NotImplementedError("write your pallas kernel here")


# speedup vs baseline: 16.4503x; 16.4503x over previous
import functools

import jax
import jax.numpy as jnp
from jax import lax
from jax.experimental import pallas as pl
from jax.experimental.pallas import tpu as pltpu
from jax.experimental.pallas import tpu_sc as plsc

NC = 2
NS = 16
L = 16
NW = NC * NS
ROWS = 128
COLS = 32768
CPW = COLS // NW
CHUNK = 256
NCHUNK = CPW // CHUNK
UNROLL = 4


def _sort16(v):
    return lax.sort(v, dimension=0)


def _rev(v):
    return lax.rev(v, dimensions=(0,))


def _column_threshold(v):
    s = [_sort16(vj) for vj in v]
    nodes = []
    for p in range(4):
        a, b = s[2 * p], s[2 * p + 1]
        rb = _rev(b)
        hi = jnp.maximum(a, rb)
        lo = jnp.minimum(a, rb)
        nodes.append((_sort16(lo), _sort16(hi)))
    tops = []
    for p in range(2):
        (a0, a1), (b0, b1) = nodes[2 * p], nodes[2 * p + 1]
        c0 = jnp.maximum(a0, _rev(b1))
        c1 = jnp.maximum(a1, _rev(b0))
        d0 = jnp.minimum(c0, c1)
        d1 = jnp.maximum(c0, c1)
        tops.append((_sort16(d0), _sort16(d1)))
    (a0, a1), (b0, b1) = tops
    e0 = jnp.maximum(a0, _rev(b1))
    e1 = jnp.maximum(a1, _rev(b0))
    return jnp.min(jnp.minimum(e0, e1))


def _make_kernel():
    mesh = plsc.VectorSubcoreMesh(
        core_axis_name="c", subcore_axis_name="s",
        num_cores=NC, num_subcores=NS)

    @functools.partial(
        pl.kernel,
        out_type=jax.ShapeDtypeStruct((ROWS, COLS), jnp.float32),
        mesh=mesh,
        scratch_types=[pltpu.VMEM((ROWS, CHUNK), jnp.float32)],
        compiler_params=pltpu.CompilerParams(
            use_tc_tiling_on_sc=False, needs_layout_passes=False),
    )
    def topk_mask(x_hbm, out_hbm, buf):
        wid = lax.axis_index("s") * NC + lax.axis_index("c")
        base_w = wid * CPW
        row_iota = lax.iota(jnp.int32, L)
        ridx = [row_iota + (L * j) for j in range(ROWS // L)]

        def do_chunk(g, carry):
            base = base_w + g * CHUNK
            pltpu.sync_copy(x_hbm.at[:, pl.ds(base, CHUNK)], buf)

            def do_cols(i, carry2):
                for u in range(UNROLL):
                    c = i * UNROLL + u
                    cidx = jnp.full((L,), 0, jnp.int32) + c
                    v = [plsc.load_gather(buf, [ridx[j], cidx])
                         for j in range(ROWS // L)]
                    t = _column_threshold(v)
                    for j in range(ROWS // L):
                        masked = jnp.where(v[j] >= t, v[j], 0.0)
                        plsc.store_scatter(buf, [ridx[j], cidx], masked)
                return carry2

            lax.fori_loop(0, CHUNK // UNROLL, do_cols, 0)
            pltpu.sync_copy(buf, out_hbm.at[:, pl.ds(base, CHUNK)])
            return carry

        lax.fori_loop(0, NCHUNK, do_chunk, 0)

    return topk_mask


_topk_mask = _make_kernel()


@jax.jit
def kernel(x):
    return _topk_mask(x)


# speedup vs baseline: 19.0516x; 1.1581x over previous
import functools

import jax
import jax.numpy as jnp
from jax import lax
from jax.experimental import pallas as pl
from jax.experimental.pallas import tpu as pltpu
from jax.experimental.pallas import tpu_sc as plsc

NC = 2
NS = 16
L = 16
NW = NC * NS
ROWS = 128
COLS = 32768
CPW = COLS // NW
CHUNK = 256
NCHUNK = CPW // CHUNK
UNROLL = 4


def _asc(v):
    return lax.sort(v, dimension=0)


def _dsc(v):
    return plsc.sort_key_val(v, v, descending=True)[0]


def _column_threshold(v):
    a = [_asc(v[2 * p]) for p in range(4)]
    d = [_dsc(v[2 * p + 1]) for p in range(4)]
    nodes = []
    for p in range(4):
        hi = jnp.maximum(a[p], d[p])
        lo = jnp.minimum(a[p], d[p])
        if p < 2:
            nodes.append((_asc(lo), _asc(hi)))
        else:
            nodes.append((_dsc(hi), _dsc(lo)))

    def top32(asc_node, dsc_node):
        (a0, a1), (b0, b1) = asc_node, dsc_node
        c0 = jnp.maximum(a0, b0)
        c1 = jnp.maximum(a1, b1)
        return jnp.minimum(c0, c1), jnp.maximum(c0, c1)

    d0, d1 = top32(nodes[0], nodes[2])
    xn = (_asc(d0), _asc(d1))
    e0, e1 = top32(nodes[1], nodes[3])
    yn = (_dsc(e1), _dsc(e0))
    f0 = jnp.maximum(xn[0], yn[0])
    f1 = jnp.maximum(xn[1], yn[1])
    return jnp.min(jnp.minimum(f0, f1))


def _make_kernel():
    mesh = plsc.VectorSubcoreMesh(
        core_axis_name="c", subcore_axis_name="s",
        num_cores=NC, num_subcores=NS)

    @functools.partial(
        pl.kernel,
        out_type=jax.ShapeDtypeStruct((ROWS, COLS), jnp.float32),
        mesh=mesh,
        scratch_types=[pltpu.VMEM((ROWS, CHUNK), jnp.float32)],
        compiler_params=pltpu.CompilerParams(
            use_tc_tiling_on_sc=False, needs_layout_passes=False),
    )
    def topk_mask(x_hbm, out_hbm, buf):
        wid = lax.axis_index("s") * NC + lax.axis_index("c")
        base_w = wid * CPW
        row_iota = lax.iota(jnp.int32, L)
        ridx = [row_iota + (L * j) for j in range(ROWS // L)]

        def do_chunk(g, carry):
            base = base_w + g * CHUNK
            pltpu.sync_copy(x_hbm.at[:, pl.ds(base, CHUNK)], buf)

            @plsc.parallel_loop(0, CHUNK, 1, unroll=UNROLL)
            def do_cols(c):
                cidx = jnp.full((L,), 0, jnp.int32) + c
                v = [plsc.load_gather(buf, [ridx[j], cidx])
                     for j in range(ROWS // L)]
                t = _column_threshold(v)
                for j in range(ROWS // L):
                    masked = jnp.where(v[j] >= t, v[j], 0.0)
                    plsc.store_scatter(buf, [ridx[j], cidx], masked)

            pltpu.sync_copy(buf, out_hbm.at[:, pl.ds(base, CHUNK)])
            return carry

        lax.fori_loop(0, NCHUNK, do_chunk, 0)

    return topk_mask


_topk_mask = _make_kernel()


@jax.jit
def kernel(x):
    return _topk_mask(x)


# speedup vs baseline: 42.0357x; 2.2064x over previous
import functools

import jax
import jax.numpy as jnp
from jax import lax
from jax.experimental import pallas as pl
from jax.experimental.pallas import tpu as pltpu
from jax.experimental.pallas import tpu_sc as plsc

NC = 2
NS = 16
L = 16
NW = NC * NS
ROWS = 128
COLS = 32768
CPW = COLS // NW
CHUNK = 256
NCHUNK = CPW // CHUNK
NGROUP = CHUNK // L


def _ce(lst, i, j):
    a, b = lst[i], lst[j]
    lst[i] = jnp.minimum(a, b)
    lst[j] = jnp.maximum(a, b)


def _oddeven_merge(lst, lo, n, r):
    step = r * 2
    if step < n:
        _oddeven_merge(lst, lo, n, step)
        _oddeven_merge(lst, lo + r, n, step)
        for i in range(lo + r, lo + n - r, step):
            _ce(lst, i, i + r)
    else:
        _ce(lst, lo, lo + r)


def _oddeven_sort(lst, lo, n):
    if n > 1:
        m = n // 2
        _oddeven_sort(lst, lo, m)
        _oddeven_sort(lst, lo + m, m)
        _oddeven_merge(lst, lo, n, 1)


def _bitonic_clean(lst, lo, n):
    if n > 1:
        m = n // 2
        for i in range(lo, lo + m):
            _ce(lst, i, i + m)
        _bitonic_clean(lst, lo, m)
        _bitonic_clean(lst, lo + m, m)


def _top32_sorted(a, b):
    t = [jnp.maximum(a[i], b[31 - i]) for i in range(32)]
    _bitonic_clean(t, 0, 32)
    return t


def _group_threshold(load_row):
    def sorted16(i0):
        blk = [load_row(i0 + t) for t in range(16)]
        _oddeven_sort(blk, 0, 16)
        return blk

    def sorted32(i0):
        blk = sorted16(i0) + sorted16(i0 + 16)
        _oddeven_merge(blk, 0, 32, 1)
        return blk

    x32 = _top32_sorted(sorted32(0), sorted32(32))
    y32 = _top32_sorted(sorted32(64), sorted32(96))
    f = [jnp.maximum(x32[i], y32[31 - i]) for i in range(32)]
    while len(f) > 1:
        f = [jnp.minimum(f[2 * i], f[2 * i + 1]) for i in range(len(f) // 2)]
    return f[0]


def _make_kernel():
    mesh = plsc.VectorSubcoreMesh(
        core_axis_name="c", subcore_axis_name="s",
        num_cores=NC, num_subcores=NS)

    @functools.partial(
        pl.kernel,
        out_type=jax.ShapeDtypeStruct((ROWS, COLS), jnp.float32),
        mesh=mesh,
        scratch_types=[pltpu.VMEM((ROWS, CHUNK), jnp.float32)],
        compiler_params=pltpu.CompilerParams(
            use_tc_tiling_on_sc=False, needs_layout_passes=False),
    )
    def topk_mask(x_hbm, out_hbm, buf):
        wid = lax.axis_index("s") * NC + lax.axis_index("c")
        base_w = wid * CPW

        def do_chunk(g, carry):
            base = base_w + g * CHUNK
            pltpu.sync_copy(x_hbm.at[:, pl.ds(base, CHUNK)], buf)

            @plsc.parallel_loop(0, NGROUP, 1)
            def do_group(g2):
                c0 = pl.multiple_of(g2 * L, L)
                t = _group_threshold(lambda r: buf[r, pl.ds(c0, L)])
                for r in range(ROWS):
                    v = buf[r, pl.ds(c0, L)]
                    buf[r, pl.ds(c0, L)] = jnp.where(v >= t, v, 0.0)

            pltpu.sync_copy(buf, out_hbm.at[:, pl.ds(base, CHUNK)])
            return carry

        lax.fori_loop(0, NCHUNK, do_chunk, 0)

    return topk_mask


_topk_mask = _make_kernel()


@jax.jit
def kernel(x):
    return _topk_mask(x)


# speedup vs baseline: 65.1423x; 1.5497x over previous
import functools

import jax
import jax.numpy as jnp
from jax import lax
from jax.experimental import pallas as pl
from jax.experimental.pallas import tpu as pltpu
from jax.experimental.pallas import tpu_sc as plsc

NC = 2
NS = 16
L = 16
NW = NC * NS
ROWS = 128
COLS = 32768
CPW = COLS // NW
CHUNK = 256
NCHUNK = CPW // CHUNK
NGROUP = CHUNK // L


def _ce(lst, i, j):
    a, b = lst[i], lst[j]
    lst[i] = jnp.minimum(a, b)
    lst[j] = jnp.maximum(a, b)


def _oddeven_merge(lst, lo, n, r):
    step = r * 2
    if step < n:
        _oddeven_merge(lst, lo, n, step)
        _oddeven_merge(lst, lo + r, n, step)
        for i in range(lo + r, lo + n - r, step):
            _ce(lst, i, i + r)
    else:
        _ce(lst, lo, lo + r)


def _oddeven_sort(lst, lo, n):
    if n > 1:
        m = n // 2
        _oddeven_sort(lst, lo, m)
        _oddeven_sort(lst, lo + m, m)
        _oddeven_merge(lst, lo, n, 1)


def _bitonic_clean(lst, lo, n):
    if n > 1:
        m = n // 2
        for i in range(lo, lo + m):
            _ce(lst, i, i + m)
        _bitonic_clean(lst, lo, m)
        _bitonic_clean(lst, lo + m, m)


def _top32_sorted(a, b):
    t = [jnp.maximum(a[i], b[31 - i]) for i in range(32)]
    _bitonic_clean(t, 0, 32)
    return t


def _group_threshold(load_row):
    def sorted16(i0):
        blk = [load_row(i0 + t) for t in range(16)]
        _oddeven_sort(blk, 0, 16)
        return blk

    def sorted32(i0):
        blk = sorted16(i0) + sorted16(i0 + 16)
        _oddeven_merge(blk, 0, 32, 1)
        return blk

    x32 = _top32_sorted(sorted32(0), sorted32(32))
    y32 = _top32_sorted(sorted32(64), sorted32(96))
    f = [jnp.maximum(x32[i], y32[31 - i]) for i in range(32)]
    while len(f) > 1:
        f = [jnp.minimum(f[2 * i], f[2 * i + 1]) for i in range(len(f) // 2)]
    return f[0]


def _make_kernel():
    mesh = plsc.VectorSubcoreMesh(
        core_axis_name="c", subcore_axis_name="s",
        num_cores=NC, num_subcores=NS)

    @functools.partial(
        pl.kernel,
        out_type=jax.ShapeDtypeStruct((ROWS, COLS), jnp.float32),
        mesh=mesh,
        scratch_types=[pltpu.VMEM((ROWS, CHUNK), jnp.float32)],
        compiler_params=pltpu.CompilerParams(
            use_tc_tiling_on_sc=True, needs_layout_passes=False),
    )
    def topk_mask(x_hbm, out_hbm, buf):
        wid = lax.axis_index("s") * NC + lax.axis_index("c")
        base_w = wid * CPW

        def do_chunk(g, carry):
            base = base_w + g * CHUNK
            pltpu.sync_copy(x_hbm.at[:, pl.ds(base, CHUNK)], buf)

            @plsc.parallel_loop(0, NGROUP, 1)
            def do_group(g2):
                c0 = pl.multiple_of(g2 * L, L)
                t = _group_threshold(lambda r: buf[r, pl.ds(c0, L)])
                for r in range(ROWS):
                    v = buf[r, pl.ds(c0, L)]
                    buf[r, pl.ds(c0, L)] = jnp.where(v >= t, v, 0.0)

            pltpu.sync_copy(buf, out_hbm.at[:, pl.ds(base, CHUNK)])
            return carry

        lax.fori_loop(0, NCHUNK, do_chunk, 0)

    return topk_mask


_topk_mask = _make_kernel()


@jax.jit
def kernel(x):
    return _topk_mask(x)


# speedup vs baseline: 65.2569x; 1.0018x over previous
import functools

import jax
import jax.numpy as jnp
from jax import lax
from jax.experimental import pallas as pl
from jax.experimental.pallas import tpu as pltpu
from jax.experimental.pallas import tpu_sc as plsc

NC = 2
NS = 16
L = 16
NW = NC * NS
ROWS = 128
COLS = 32768
CPW = COLS // NW
CHUNK = 256
NCHUNK = CPW // CHUNK
NGROUP = CHUNK // L


def _ce(lst, i, j):
    a, b = lst[i], lst[j]
    lst[i] = jnp.minimum(a, b)
    lst[j] = jnp.maximum(a, b)


def _oddeven_merge(lst, lo, n, r):
    step = r * 2
    if step < n:
        _oddeven_merge(lst, lo, n, step)
        _oddeven_merge(lst, lo + r, n, step)
        for i in range(lo + r, lo + n - r, step):
            _ce(lst, i, i + r)
    else:
        _ce(lst, lo, lo + r)


def _oddeven_sort(lst, lo, n):
    if n > 1:
        m = n // 2
        _oddeven_sort(lst, lo, m)
        _oddeven_sort(lst, lo + m, m)
        _oddeven_merge(lst, lo, n, 1)


def _bitonic_clean(lst, lo, n):
    if n > 1:
        m = n // 2
        for i in range(lo, lo + m):
            _ce(lst, i, i + m)
        _bitonic_clean(lst, lo, m)
        _bitonic_clean(lst, lo + m, m)


def _top32_sorted(a, b):
    t = [jnp.maximum(a[i], b[31 - i]) for i in range(32)]
    _bitonic_clean(t, 0, 32)
    return t


def _group_threshold(load_row):
    def sorted16(i0):
        blk = [load_row(i0 + t) for t in range(16)]
        _oddeven_sort(blk, 0, 16)
        return blk

    def sorted32(i0):
        blk = sorted16(i0) + sorted16(i0 + 16)
        _oddeven_merge(blk, 0, 32, 1)
        return blk

    x32 = _top32_sorted(sorted32(0), sorted32(32))
    y32 = _top32_sorted(sorted32(64), sorted32(96))
    f = [jnp.maximum(x32[i], y32[31 - i]) for i in range(32)]
    while len(f) > 1:
        f = [jnp.minimum(f[2 * i], f[2 * i + 1]) for i in range(len(f) // 2)]
    return f[0]


def _make_kernel():
    mesh = plsc.VectorSubcoreMesh(
        core_axis_name="c", subcore_axis_name="s",
        num_cores=NC, num_subcores=NS)

    NBUF = 3

    @functools.partial(
        pl.kernel,
        out_type=jax.ShapeDtypeStruct((ROWS, COLS), jnp.float32),
        mesh=mesh,
        scratch_types=(
            [pltpu.VMEM((ROWS, CHUNK), jnp.float32)] * NBUF
            + [pltpu.SemaphoreType.DMA] * (2 * NBUF)),
        compiler_params=pltpu.CompilerParams(
            use_tc_tiling_on_sc=True, needs_layout_passes=False),
    )
    def topk_mask(x_hbm, out_hbm, *scratch):
        bufs = scratch[:NBUF]
        sin = scratch[NBUF:2 * NBUF]
        sout = scratch[2 * NBUF:]
        wid = lax.axis_index("s") * NC + lax.axis_index("c")
        base_w = wid * CPW

        def copy_in(g):
            base = base_w + g * CHUNK
            return pltpu.async_copy(
                x_hbm.at[:, pl.ds(base, CHUNK)], bufs[g % NBUF],
                sin[g % NBUF])

        def copy_out(g):
            base = base_w + g * CHUNK
            return pltpu.async_copy(
                bufs[g % NBUF], out_hbm.at[:, pl.ds(base, CHUNK)],
                sout[g % NBUF])

        def compute(g):
            buf = bufs[g % NBUF]

            @plsc.parallel_loop(0, NGROUP, 1)
            def do_group(g2):
                c0 = pl.multiple_of(g2 * L, L)
                t = _group_threshold(lambda r: buf[r, pl.ds(c0, L)])
                for r in range(ROWS):
                    v = buf[r, pl.ds(c0, L)]
                    buf[r, pl.ds(c0, L)] = jnp.where(v >= t, v, 0.0)

        d_in = {g: copy_in(g) for g in range(min(NBUF, NCHUNK))}
        d_out = {}
        for g in range(NCHUNK):
            d_in[g].wait()
            compute(g)
            d_out[g] = copy_out(g)
            nxt = g + NBUF
            if nxt < NCHUNK:
                d_out[nxt - NBUF].wait()
                d_in[nxt] = copy_in(nxt)
        for g in range(max(0, NCHUNK - NBUF), NCHUNK):
            d_out[g].wait()

    return topk_mask


_topk_mask = _make_kernel()


@jax.jit
def kernel(x):
    return _topk_mask(x)


# speedup vs baseline: 65.3657x; 1.0017x over previous
import functools

import jax
import jax.numpy as jnp
from jax import lax
from jax.experimental import pallas as pl
from jax.experimental.pallas import tpu as pltpu
from jax.experimental.pallas import tpu_sc as plsc

NC = 2
NS = 16
L = 16
NW = NC * NS
ROWS = 128
COLS = 32768
CPW = COLS // NW
CHUNK = 256
NCHUNK = CPW // CHUNK
NGROUP = CHUNK // L


def _ce(lst, i, j):
    a, b = lst[i], lst[j]
    lst[i] = jnp.minimum(a, b)
    lst[j] = jnp.maximum(a, b)


def _oddeven_merge(lst, lo, n, r):
    step = r * 2
    if step < n:
        _oddeven_merge(lst, lo, n, step)
        _oddeven_merge(lst, lo + r, n, step)
        for i in range(lo + r, lo + n - r, step):
            _ce(lst, i, i + r)
    else:
        _ce(lst, lo, lo + r)


def _oddeven_sort(lst, lo, n):
    if n > 1:
        m = n // 2
        _oddeven_sort(lst, lo, m)
        _oddeven_sort(lst, lo + m, m)
        _oddeven_merge(lst, lo, n, 1)


def _bitonic_clean(lst, lo, n):
    if n > 1:
        m = n // 2
        for i in range(lo, lo + m):
            _ce(lst, i, i + m)
        _bitonic_clean(lst, lo, m)
        _bitonic_clean(lst, lo + m, m)


def _top32_sorted(a, b):
    t = [jnp.maximum(a[i], b[31 - i]) for i in range(32)]
    _bitonic_clean(t, 0, 32)
    return t


def _group_threshold(load_row):
    def sorted16(i0):
        blk = [load_row(i0 + t) for t in range(16)]
        _oddeven_sort(blk, 0, 16)
        return blk

    def sorted32(i0):
        blk = sorted16(i0) + sorted16(i0 + 16)
        _oddeven_merge(blk, 0, 32, 1)
        return blk

    x32 = _top32_sorted(sorted32(0), sorted32(32))
    y32 = _top32_sorted(sorted32(64), sorted32(96))
    f = [jnp.maximum(x32[i], y32[31 - i]) for i in range(32)]
    while len(f) > 1:
        f = [jnp.minimum(f[2 * i], f[2 * i + 1]) for i in range(len(f) // 2)]
    return f[0]


def _make_kernel():
    mesh = plsc.VectorSubcoreMesh(
        core_axis_name="c", subcore_axis_name="s",
        num_cores=NC, num_subcores=NS)

    NBUF = 3

    @functools.partial(
        pl.kernel,
        out_type=jax.ShapeDtypeStruct((ROWS, COLS), jnp.float32),
        mesh=mesh,
        scratch_types=(
            [pltpu.VMEM((ROWS, CHUNK), jnp.float32)] * NBUF
            + [pltpu.SemaphoreType.DMA] * (2 * NBUF)),
        compiler_params=pltpu.CompilerParams(use_tc_tiling_on_sc=True),
    )
    def topk_mask(x_hbm, out_hbm, *scratch):
        bufs = scratch[:NBUF]
        sin = scratch[NBUF:2 * NBUF]
        sout = scratch[2 * NBUF:]
        wid = lax.axis_index("s") * NC + lax.axis_index("c")
        base_w = wid * CPW

        def copy_in(g):
            base = base_w + g * CHUNK
            return pltpu.async_copy(
                x_hbm.at[:, pl.ds(base, CHUNK)], bufs[g % NBUF],
                sin[g % NBUF])

        def copy_out(g):
            base = base_w + g * CHUNK
            return pltpu.async_copy(
                bufs[g % NBUF], out_hbm.at[:, pl.ds(base, CHUNK)],
                sout[g % NBUF])

        def compute(g):
            buf = bufs[g % NBUF]

            @plsc.parallel_loop(0, NGROUP, 1)
            def do_group(g2):
                c0 = pl.multiple_of(g2 * L, L)
                t = _group_threshold(lambda r: buf[r, pl.ds(c0, L)])
                for r in range(ROWS):
                    v = buf[r, pl.ds(c0, L)]
                    buf[r, pl.ds(c0, L)] = jnp.where(v >= t, v, 0.0)

        d_in = {g: copy_in(g) for g in range(min(NBUF, NCHUNK))}
        d_out = {}
        for g in range(NCHUNK):
            d_in[g].wait()
            compute(g)
            d_out[g] = copy_out(g)
            nxt = g + NBUF
            if nxt < NCHUNK:
                d_out[nxt - NBUF].wait()
                d_in[nxt] = copy_in(nxt)
        for g in range(max(0, NCHUNK - NBUF), NCHUNK):
            d_out[g].wait()

    return topk_mask


_topk_mask = _make_kernel()


@jax.jit
def kernel(x):
    return _topk_mask(x)
